# baseline (speedup 1.0000x reference)
import jax
import jax.numpy as jnp
from jax import lax
from jax.experimental import pallas as pl
from jax.experimental.pallas import tpu as pltpu
from jax.experimental.pallas import tpu_sc as plsc

_B, _J, _C = 4096, 25, 256
_REG = ((0, 0), (1, 20), (3, 3), (2, 20), (21, 21), (22, 7), (6, 5),
        (4, 20), (23, 23), (24, 11), (10, 9), (8, 20), (0, 0), (12, 13),
        (14, 15), (0, 0), (16, 17), (18, 19))
_R = len(_REG)

_PAIR_DEST = {1: 1, 3: 4, 5: 5, 6: 6, 7: 7, 9: 8, 10: 9, 11: 10,
              13: 11, 14: 12, 16: 13, 17: 14}
_PAIRS = tuple((r, a, b) for r, (a, b) in enumerate(_REG) if a != b)
_PJ = tuple(sorted({j for _, a, b in _PAIRS for j in (a, b)}))

_OUT_RUNS = ((0, 0, 2), (2, 3, 2), (4, 21, 1), (5, 5, 3), (8, 23, 1),
             (9, 8, 3), (12, 0, 1), (13, 11, 2), (15, 0, 1), (16, 13, 2))

_NC, _NS = 2, 16
_NW = _NC * _NS
_BW = _B // _NW
_CB = 8
_NCHUNK = _BW // _CB
_LANES = 16
_NLG = _C // _LANES


def _body(x_hbm, o_hbm, in0, in1, isem0, isem1, osem0, osem1):
    wid = lax.axis_index("s") * _NC + lax.axis_index("c")
    start = wid * _BW
    ins, isems, osems = (in0, in1), (isem0, isem1), (osem0, osem1)

    def in_copy(c, slot):
        return pltpu.make_async_copy(
            x_hbm.at[:, pl.ds(start + c * _CB, _CB), :], ins[slot], isems[slot])

    def out_copies(c, slot):
        cps = []
        for rs, row, ln in _OUT_RUNS:
            cps.append(pltpu.make_async_copy(
                ins[slot].at[pl.ds(row, ln)],
                o_hbm.at[pl.ds(rs, ln), pl.ds(start + c * _CB, _CB), :],
                osems[slot]))
        return cps

    def start_all(cps):
        for cp in cps:
            cp.start()

    def wait_all(cps):
        for cp in cps:
            cp.wait()

    def compute(slot):
        in_v = ins[slot]

        def batch(b, carry):
            for lg in range(_NLG):
                s = lg * _LANES
                rows = {j: in_v[j, b, pl.ds(s, _LANES)] for j in _PJ}
                for r, a, bj in _PAIRS:
                    in_v[_PAIR_DEST[r], b, pl.ds(s, _LANES)] = (
                        rows[a] + rows[bj]) * 0.5
            return carry

        lax.fori_loop(0, _CB, batch, 0)

    in_copy(0, 0).start()

    def pair_step(i, carry):
        for k in range(2):
            c = 2 * i + k
            slot, nxt = k, 1 - k

            @pl.when(c >= 1)
            def _():
                wait_all(out_copies(c - 1, nxt))

            @pl.when(c + 1 < _NCHUNK)
            def _():
                in_copy(c + 1, nxt).start()

            in_copy(c, slot).wait()
            compute(slot)
            start_all(out_copies(c, slot))
        return carry

    lax.fori_loop(0, _NCHUNK // 2, pair_step, 0)
    wait_all(out_copies(_NCHUNK - 1, 1))


@jax.jit
def kernel(x):
    xt = jnp.transpose(x, (1, 0, 2))
    mesh = plsc.VectorSubcoreMesh(core_axis_name="c", subcore_axis_name="s")
    f = pl.kernel(
        _body,
        out_type=jax.ShapeDtypeStruct((_R, _B, _C), jnp.float32),
        mesh=mesh,
        scratch_types=[
            pltpu.VMEM((_J, _CB, _C), jnp.float32),
            pltpu.VMEM((_J, _CB, _C), jnp.float32),
            pltpu.SemaphoreType.DMA,
            pltpu.SemaphoreType.DMA,
            pltpu.SemaphoreType.DMA,
            pltpu.SemaphoreType.DMA,
        ],
    )
    ot = f(xt)
    return jnp.transpose(ot, (1, 0, 2))


# speedup vs baseline: 1.0424x; 1.0424x over previous
import jax
import jax.numpy as jnp
from jax import lax
from jax.experimental import pallas as pl
from jax.experimental.pallas import tpu as pltpu
from jax.experimental.pallas import tpu_sc as plsc

_B, _J, _C = 4096, 25, 256
_REG = ((0, 0), (1, 20), (3, 3), (2, 20), (21, 21), (22, 7), (6, 5),
        (4, 20), (23, 23), (24, 11), (10, 9), (8, 20), (0, 0), (12, 13),
        (14, 15), (0, 0), (16, 17), (18, 19))
_R = len(_REG)
_NC, _NS = 2, 16
_NW = _NC * _NS
_BW = _B // _NW
_CB = 8
_NCHUNK = _BW // _CB
_LANES = 16
_HC = _C // 2
_NLG = _HC // _LANES


def _body(x_hbm, o_hbm, in0, in1, out0, out1, isem0, isem1, osem0, osem1):
    wid = lax.axis_index("s") * _NC + lax.axis_index("c")
    start = wid * _BW
    ins, outs, isems, osems = (in0, in1), (out0, out1), (isem0, isem1), (osem0, osem1)

    def in_copy(c, half, slot):
        return pltpu.make_async_copy(
            x_hbm.at[:, pl.ds(start + c * _CB, _CB), pl.ds(half * _HC, _HC)],
            ins[slot], isems[slot])

    def out_copy(c, half, slot):
        return pltpu.make_async_copy(
            outs[slot],
            o_hbm.at[:, pl.ds(start + c * _CB, _CB), pl.ds(half * _HC, _HC)],
            osems[slot])

    def compute(slot):
        in_v, out_v = ins[slot], outs[slot]

        def batch(b, carry):
            for lg in range(_NLG):
                s = lg * _LANES
                rows = [in_v[j, b, pl.ds(s, _LANES)] for j in range(_J)]
                for r in range(_R):
                    j0, j1 = _REG[r]
                    out_v[r, b, pl.ds(s, _LANES)] = (rows[j0] + rows[j1]) * 0.5
            return carry

        lax.fori_loop(0, _CB, batch, 0)

    in_copy(0, 0, 0).start()

    def chunk(c, carry):
        for half in range(2):
            slot = half
            nxt = 1 - half
            if half == 0:
                in_copy(c, 1, nxt).start()
            else:
                @pl.when(c + 1 < _NCHUNK)
                def _():
                    in_copy(c + 1, 0, nxt).start()

            in_copy(c, half, slot).wait()

            @pl.when(2 * c + half >= 2)
            def _():
                pc = c - 1 + half
                out_copy(pc, half, slot).wait()

            compute(slot)
            out_copy(c, half, slot).start()
        return carry

    lax.fori_loop(0, _NCHUNK, chunk, 0)
    out_copy(_NCHUNK - 1, 0, 0).wait()
    out_copy(_NCHUNK - 1, 1, 1).wait()


@jax.jit
def kernel(x):
    xt = jnp.transpose(x, (1, 0, 2))
    mesh = plsc.VectorSubcoreMesh(core_axis_name="c", subcore_axis_name="s")
    f = pl.kernel(
        _body,
        out_type=jax.ShapeDtypeStruct((_R, _B, _C), jnp.float32),
        mesh=mesh,
        scratch_types=[
            pltpu.VMEM((_J, _CB, _HC), jnp.float32),
            pltpu.VMEM((_J, _CB, _HC), jnp.float32),
            pltpu.VMEM((_R, _CB, _HC), jnp.float32),
            pltpu.VMEM((_R, _CB, _HC), jnp.float32),
            pltpu.SemaphoreType.DMA,
            pltpu.SemaphoreType.DMA,
            pltpu.SemaphoreType.DMA,
            pltpu.SemaphoreType.DMA,
        ],
    )
    ot = f(xt)
    return jnp.transpose(ot, (1, 0, 2))


# speedup vs baseline: 1.0563x; 1.0133x over previous
import jax
import jax.numpy as jnp
from jax import lax
from jax.experimental import pallas as pl
from jax.experimental.pallas import tpu as pltpu
from jax.experimental.pallas import tpu_sc as plsc

_B, _J, _C = 4096, 25, 256
_REG = ((0, 0), (1, 20), (3, 3), (2, 20), (21, 21), (22, 7), (6, 5),
        (4, 20), (23, 23), (24, 11), (10, 9), (8, 20), (0, 0), (12, 13),
        (14, 15), (0, 0), (16, 17), (18, 19))
_R = len(_REG)
_NC, _NS = 2, 16
_NW = _NC * _NS
_BW = _B // _NW
_CB = 4
_NCHUNK = _BW // _CB
_LANES = 16
_NLG = _C // _LANES


def _body(x_hbm, o_hbm, in0, in1, out0, out1, isem0, isem1, osem0, osem1):
    wid = lax.axis_index("s") * _NC + lax.axis_index("c")
    start = wid * _BW
    ins, outs = (in0, in1), (out0, out1)
    isems, osems = (isem0, isem1), (osem0, osem1)

    def in_copy(c, slot):
        return pltpu.make_async_copy(
            x_hbm.at[:, pl.ds(start + c * _CB, _CB), :], ins[slot], isems[slot])

    def out_copy(c, slot):
        return pltpu.make_async_copy(
            outs[slot], o_hbm.at[:, pl.ds(start + c * _CB, _CB), :],
            osems[slot])

    def compute(slot):
        in_v, out_v = ins[slot], outs[slot]

        def batch(b, carry):
            for lg in range(_NLG):
                s = lg * _LANES
                rows = [in_v[j, b, pl.ds(s, _LANES)] for j in range(_J)]
                for r in range(_R):
                    j0, j1 = _REG[r]
                    out_v[r, b, pl.ds(s, _LANES)] = (rows[j0] + rows[j1]) * 0.5
            return carry

        lax.fori_loop(0, _CB, batch, 0)

    in_copy(0, 0).start()

    def pair_step(i, carry):
        for k in range(2):
            c = 2 * i + k
            slot, nxt = k, 1 - k

            @pl.when(c + 1 < _NCHUNK)
            def _():
                in_copy(c + 1, nxt).start()

            in_copy(c, slot).wait()

            @pl.when(c >= 2)
            def _():
                out_copy(c - 2, slot).wait()

            compute(slot)
            out_copy(c, slot).start()
        return carry

    lax.fori_loop(0, _NCHUNK // 2, pair_step, 0)
    out_copy(_NCHUNK - 2, 0).wait()
    out_copy(_NCHUNK - 1, 1).wait()


@jax.jit
def kernel(x):
    xt = jnp.transpose(x, (1, 0, 2))
    mesh = plsc.VectorSubcoreMesh(core_axis_name="c", subcore_axis_name="s")
    f = pl.kernel(
        _body,
        out_type=jax.ShapeDtypeStruct((_R, _B, _C), jnp.float32),
        mesh=mesh,
        scratch_types=[
            pltpu.VMEM((_J, _CB, _C), jnp.float32),
            pltpu.VMEM((_J, _CB, _C), jnp.float32),
            pltpu.VMEM((_R, _CB, _C), jnp.float32),
            pltpu.VMEM((_R, _CB, _C), jnp.float32),
            pltpu.SemaphoreType.DMA,
            pltpu.SemaphoreType.DMA,
            pltpu.SemaphoreType.DMA,
            pltpu.SemaphoreType.DMA,
        ],
    )
    ot = f(xt)
    return jnp.transpose(ot, (1, 0, 2))


# speedup vs baseline: 1.0570x; 1.0007x over previous
import jax
import jax.numpy as jnp
from jax import lax
from jax.experimental import pallas as pl
from jax.experimental.pallas import tpu as pltpu
from jax.experimental.pallas import tpu_sc as plsc

_B, _J, _C = 4096, 25, 256
_REG = ((0, 0), (1, 20), (3, 3), (2, 20), (21, 21), (22, 7), (6, 5),
        (4, 20), (23, 23), (24, 11), (10, 9), (8, 20), (0, 0), (12, 13),
        (14, 15), (0, 0), (16, 17), (18, 19))
_R = len(_REG)
_NC, _NS = 2, 16
_NW = _NC * _NS
_BW = _B // _NW
_CB = 4
_NCHUNK = _BW // _CB
_LANES = 16
_NLG = _C // _LANES


def _body(x_hbm, o_hbm, in0, in1, out0, out1, isem0, isem1, osem0, osem1):
    wid = lax.axis_index("s") * _NC + lax.axis_index("c")
    start = wid * _BW
    ins, outs = (in0, in1), (out0, out1)
    isems, osems = (isem0, isem1), (osem0, osem1)

    def in_copy(c, slot):
        return pltpu.make_async_copy(
            x_hbm.at[:, pl.ds(start + c * _CB, _CB), :], ins[slot], isems[slot])

    def out_copy(c, slot):
        return pltpu.make_async_copy(
            outs[slot], o_hbm.at[:, pl.ds(start + c * _CB, _CB), :],
            osems[slot])

    def compute(slot):
        in_v, out_v = ins[slot], outs[slot]

        def batch(b, carry):
            for lg in range(_NLG):
                s = lg * _LANES
                rows = [in_v[j, b, pl.ds(s, _LANES)] for j in range(_J)]
                for r in range(_R):
                    j0, j1 = _REG[r]
                    if j0 == j1:
                        out_v[r, b, pl.ds(s, _LANES)] = rows[j0]
                    else:
                        out_v[r, b, pl.ds(s, _LANES)] = (
                            rows[j0] + rows[j1]) * 0.5
            return carry

        lax.fori_loop(0, _CB, batch, 0)

    in_copy(0, 0).start()

    def pair_step(i, carry):
        for k in range(2):
            c = 2 * i + k
            slot, nxt = k, 1 - k

            @pl.when(c + 1 < _NCHUNK)
            def _():
                in_copy(c + 1, nxt).start()

            in_copy(c, slot).wait()

            @pl.when(c >= 2)
            def _():
                out_copy(c - 2, slot).wait()

            compute(slot)
            out_copy(c, slot).start()
        return carry

    lax.fori_loop(0, _NCHUNK // 2, pair_step, 0)
    out_copy(_NCHUNK - 2, 0).wait()
    out_copy(_NCHUNK - 1, 1).wait()


@jax.jit
def kernel(x):
    xt = jnp.transpose(x, (1, 0, 2))
    mesh = plsc.VectorSubcoreMesh(core_axis_name="c", subcore_axis_name="s")
    f = pl.kernel(
        _body,
        out_type=jax.ShapeDtypeStruct((_R, _B, _C), jnp.float32),
        mesh=mesh,
        scratch_types=[
            pltpu.VMEM((_J, _CB, _C), jnp.float32),
            pltpu.VMEM((_J, _CB, _C), jnp.float32),
            pltpu.VMEM((_R, _CB, _C), jnp.float32),
            pltpu.VMEM((_R, _CB, _C), jnp.float32),
            pltpu.SemaphoreType.DMA,
            pltpu.SemaphoreType.DMA,
            pltpu.SemaphoreType.DMA,
            pltpu.SemaphoreType.DMA,
        ],
    )
    ot = f(xt)
    return jnp.transpose(ot, (1, 0, 2))
